# baseline (speedup 1.0000x reference)
import functools

import jax
import jax.numpy as jnp
from jax import lax
from jax.experimental import pallas as pl
from jax.experimental.pallas import tpu as pltpu
from jax.experimental.pallas import tpu_sc as plsc

VOC_DIM = 64
NC, NS = 2, 16
NW = NC * NS
LANES = 16
PBLK = 1024


def _project_pack(table_t, W1):
    V = table_t.shape[1]
    nblk = (V + 2 * PBLK - 1) // (2 * PBLK)
    vblk = V // PBLK

    def body(w_ref, a_ref, b_ref, o_ref):
        dn = (((0,), (1,)), ((), ()))
        pa = lax.dot_general(a_ref[...], w_ref[...], dn,
                             preferred_element_type=jnp.float32)
        pb = lax.dot_general(b_ref[...], w_ref[...], dn,
                             preferred_element_type=jnp.float32)
        o_ref[...] = jnp.concatenate([pa, pb], axis=1)

    return pl.pallas_call(
        body,
        grid=(nblk,),
        in_specs=[
            pl.BlockSpec((VOC_DIM, VOC_DIM), lambda i: (0, 0)),
            pl.BlockSpec((VOC_DIM, PBLK), lambda i: (0, 2 * i)),
            pl.BlockSpec(
                (VOC_DIM, PBLK),
                lambda i: (0, jnp.minimum(2 * i + 1, vblk - 1))),
        ],
        out_specs=pl.BlockSpec((PBLK, 2 * VOC_DIM), lambda i: (i, 0)),
        out_shape=jax.ShapeDtypeStruct((nblk * PBLK, 2 * VOC_DIM),
                                       jnp.float32),
    )(W1, table_t, table_t)


def _gather_sum(B):
    bpw = B // NW
    mesh = plsc.VectorSubcoreMesh(core_axis_name="c", subcore_axis_name="s")

    @functools.partial(
        pl.kernel,
        out_type=jax.ShapeDtypeStruct((B * VOC_DIM,), jnp.float32),
        mesh=mesh,
        scratch_types=[
            pltpu.SMEM((bpw,), jnp.int32),
            pltpu.SMEM((bpw,), jnp.int32),
            pltpu.SMEM((bpw,), jnp.int32),
            pltpu.SMEM((bpw,), jnp.int32),
            pltpu.VMEM((bpw,), jnp.int32),
            pltpu.VMEM((bpw * VOC_DIM,), jnp.float32),
            pltpu.VMEM((bpw, 2 * VOC_DIM), jnp.float32),
            pltpu.VMEM((bpw, 2 * VOC_DIM), jnp.float32),
            pltpu.VMEM((bpw, 2 * VOC_DIM), jnp.float32),
            pltpu.SemaphoreType.DMA,
            pltpu.SemaphoreType.DMA,
            pltpu.SemaphoreType.DMA,
        ],
    )
    def gather_sum(x1h, x2h, x4h, x5h, tableh, outh,
                   idx_s, p1_s, p2_s, p3_s, idx_v,
                   accv, g1, g2, g3, s1, s2, s3):
        wid = lax.axis_index("s") * NC + lax.axis_index("c")
        base = wid * bpw
        sl = pl.ds(base, bpw)

        def fire(xh, dstv, par_s, sem):
            pltpu.sync_copy(xh.at[sl], idx_v)

            def stage16(t, _):
                v = idx_v[pl.ds(t * LANES, LANES)]
                for lane in range(LANES):
                    idx_s[t * LANES + lane] = v[lane]
                return 0
            lax.fori_loop(0, bpw // LANES, stage16, 0)

            def body(j, _):
                i = idx_s[j]
                r = i & 2047
                par_s[j] = r >> 10
                row = ((i >> 11) << 10) | (r & 1023)
                pltpu.async_copy(
                    tableh.at[pl.ds(row, 1), :],
                    dstv.at[pl.ds(j, 1), :], sem)
                return 0
            lax.fori_loop(0, bpw, body, 0)

        def drain(dstv, sem):
            pltpu.make_async_copy(
                tableh.at[pl.ds(0, bpw), :], dstv, sem).wait()

        def accum(src, par_s, init):
            def body(r, _):
                p = par_s[r]
                for half in range(2):
                    @pl.when(p == half)
                    def _():
                        for c in range(VOC_DIM // LANES):
                            s = pl.ds(r * VOC_DIM + c * LANES, LANES)
                            v = src[r, pl.ds(half * VOC_DIM + c * LANES,
                                             LANES)]
                            if init:
                                accv[s] = v
                            else:
                                plsc.addupdate(accv.at[s], v)
                return 0
            lax.fori_loop(0, bpw, body, 0)

        fire(x1h, g1, p1_s, s1)
        fire(x2h, g2, p2_s, s2)
        fire(x4h, g3, p3_s, s3)
        drain(g1, s1)
        accum(g1, p1_s, init=True)
        fire(x5h, g1, p1_s, s1)
        drain(g2, s2)
        accum(g2, p2_s, init=False)
        drain(g3, s3)
        accum(g3, p3_s, init=False)
        drain(g1, s1)
        accum(g1, p1_s, init=False)
        pltpu.sync_copy(
            accv, outh.at[pl.ds(base * VOC_DIM, bpw * VOC_DIM)])

    return gather_sum


def kernel(x1, x2, x4, x5, codebook, W1):
    B = x1.shape[0]
    table = _project_pack(codebook.T, W1)
    h = B // 2
    gs = _gather_sum(h)
    s0 = gs(x1[:h], x2[:h], x4[:h], x5[:h], table)
    s1 = gs(x1[h:], x2[h:], x4[h:], x5[h:], table)
    return jnp.concatenate([s0, s1]).reshape(B, VOC_DIM)


# speedup vs baseline: 1.4040x; 1.4040x over previous
import functools

import jax
import jax.numpy as jnp
from jax import lax
from jax.experimental import pallas as pl
from jax.experimental.pallas import tpu as pltpu
from jax.experimental.pallas import tpu_sc as plsc

VOC_DIM = 64
NC, NS = 2, 16
NW = NC * NS
LANES = 16

def _project(table_t, W1):
    V = table_t.shape[1]
    blk = 4096
    nblk = (V + blk - 1) // blk

    def body(w_ref, a_ref, o_ref):
        o_ref[...] = lax.dot_general(
            a_ref[...], w_ref[...], (((0,), (1,)), ((), ())),
            preferred_element_type=jnp.float32)

    return pl.pallas_call(
        body,
        grid=(nblk,),
        in_specs=[
            pl.BlockSpec((VOC_DIM, VOC_DIM), lambda i: (0, 0)),
            pl.BlockSpec((VOC_DIM, blk), lambda i: (0, i)),
        ],
        out_specs=pl.BlockSpec((blk, VOC_DIM), lambda i: (i, 0)),
        out_shape=jax.ShapeDtypeStruct((nblk * blk, VOC_DIM), jnp.float32),
    )(W1, table_t)


def _gather_sum(B):
    bpw = B // NW
    mesh = plsc.VectorSubcoreMesh(core_axis_name="c", subcore_axis_name="s")

    @functools.partial(
        pl.kernel,
        out_type=jax.ShapeDtypeStruct((B, VOC_DIM), jnp.float32),
        mesh=mesh,
        scratch_types=[
            pltpu.SMEM((bpw,), jnp.int32),
            pltpu.VMEM((bpw,), jnp.int32),
            pltpu.VMEM((bpw, VOC_DIM), jnp.float32),
            pltpu.VMEM((bpw, VOC_DIM), jnp.float32),
            pltpu.VMEM((bpw, VOC_DIM), jnp.float32),
            pltpu.SemaphoreType.DMA,
            pltpu.SemaphoreType.DMA,
            pltpu.SemaphoreType.DMA,
        ],
    )
    def gather_sum(x1h, x2h, x4h, x5h, tableh, outh,
                   idx_s, idx_v, accv, g2, g3, s1, s2, s3):
        wid = lax.axis_index("s") * NC + lax.axis_index("c")
        base = wid * bpw
        sl = pl.ds(base, bpw)

        def fire(xh, dstv, sem):
            pltpu.sync_copy(xh.at[sl], idx_v)

            def stage16(t, _):
                v = idx_v[pl.ds(t * LANES, LANES)]
                for lane in range(LANES):
                    idx_s[t * LANES + lane] = v[lane]
                return 0
            lax.fori_loop(0, bpw // LANES, stage16, 0)

            def body(j, _):
                i = idx_s[j]
                pltpu.async_copy(
                    tableh.at[pl.ds(i, 1), :],
                    dstv.at[pl.ds(j, 1), :], sem)
                return 0
            lax.fori_loop(0, bpw, body, 0)

        def drain(dstv, sem):
            pltpu.make_async_copy(
                tableh.at[pl.ds(0, bpw), :], dstv, sem).wait()

        def accum(src):
            def body(r, _):
                for c in range(VOC_DIM // LANES):
                    s = pl.ds(c * LANES, LANES)
                    plsc.addupdate(accv.at[r, s], src[r, s])
                return 0
            lax.fori_loop(0, bpw, body, 0)

        fire(x1h, accv, s1)
        fire(x2h, g2, s2)
        fire(x4h, g3, s3)
        drain(accv, s1)
        drain(g2, s2)
        accum(g2)
        drain(g3, s3)
        fire(x5h, g2, s2)
        accum(g3)
        drain(g2, s2)
        accum(g2)
        pltpu.sync_copy(accv, outh.at[sl])

    return gather_sum


def kernel(x1, x2, x4, x5, codebook, W1):
    B = x1.shape[0]
    table = _project(codebook.T, W1)
    h = B // 2
    gs = _gather_sum(h)
    s0 = gs(x1[:h], x2[:h], x4[:h], x5[:h], table)
    s1 = gs(x1[h:], x2[h:], x4[h:], x5[h:], table)
    return jnp.concatenate([s0, s1], axis=0)


# speedup vs baseline: 1.8415x; 1.3117x over previous
import functools

import jax
import jax.numpy as jnp
from jax import lax
from jax.experimental import pallas as pl
from jax.experimental.pallas import tpu as pltpu
from jax.experimental.pallas import tpu_sc as plsc

VOC_DIM = 64
NC, NS = 2, 16
NW = NC * NS
LANES = 16

def _project(table_t, W1):
    V = table_t.shape[1]
    blk = 16384
    nblk = (V + blk - 1) // blk

    def body(w_ref, a_ref, o_ref):
        o_ref[...] = lax.dot_general(
            a_ref[...], w_ref[...], (((0,), (1,)), ((), ())),
            preferred_element_type=jnp.float32)

    return pl.pallas_call(
        body,
        grid=(nblk,),
        compiler_params=pltpu.CompilerParams(
            fuse_transposed_lhs_in_matmul=True),
        in_specs=[
            pl.BlockSpec((VOC_DIM, VOC_DIM), lambda i: (0, 0)),
            pl.BlockSpec((VOC_DIM, blk), lambda i: (0, i)),
        ],
        out_specs=pl.BlockSpec((blk, VOC_DIM), lambda i: (i, 0)),
        out_shape=jax.ShapeDtypeStruct((nblk * blk, VOC_DIM), jnp.float32),
    )(W1, table_t)


def _gather_sum(B):
    bpw = B // NW
    mesh = plsc.VectorSubcoreMesh(core_axis_name="c", subcore_axis_name="s")

    @functools.partial(
        pl.kernel,
        out_type=jax.ShapeDtypeStruct((B, VOC_DIM), jnp.float32),
        mesh=mesh,
        scratch_types=[
            pltpu.SMEM((bpw,), jnp.int32),
            pltpu.VMEM((bpw,), jnp.int32),
            pltpu.VMEM((bpw, VOC_DIM), jnp.float32),
            pltpu.VMEM((bpw, VOC_DIM), jnp.float32),
            pltpu.VMEM((bpw, VOC_DIM), jnp.float32),
            pltpu.SemaphoreType.DMA,
            pltpu.SemaphoreType.DMA,
            pltpu.SemaphoreType.DMA,
        ],
    )
    def gather_sum(x1h, x2h, x4h, x5h, tableh, outh,
                   idx_s, idx_v, accv, g2, g3, s1, s2, s3):
        wid = lax.axis_index("s") * NC + lax.axis_index("c")
        base = wid * bpw
        sl = pl.ds(base, bpw)

        def fire(xh, dstv, sem):
            pltpu.sync_copy(xh.at[sl], idx_v)

            def stage16(t, _):
                v = idx_v[pl.ds(t * LANES, LANES)]
                for lane in range(LANES):
                    idx_s[t * LANES + lane] = v[lane]
                return 0
            lax.fori_loop(0, bpw // LANES, stage16, 0)

            def body(j, _):
                i = idx_s[j]
                pltpu.async_copy(
                    tableh.at[pl.ds(i, 1), :],
                    dstv.at[pl.ds(j, 1), :], sem)
                return 0
            lax.fori_loop(0, bpw, body, 0)

        def drain(dstv, sem):
            pltpu.make_async_copy(
                tableh.at[pl.ds(0, bpw), :], dstv, sem).wait()

        def accum(src):
            def body(r, _):
                for c in range(VOC_DIM // LANES):
                    s = pl.ds(c * LANES, LANES)
                    plsc.addupdate(accv.at[r, s], src[r, s])
                return 0
            lax.fori_loop(0, bpw, body, 0)

        fire(x1h, accv, s1)
        fire(x2h, g2, s2)
        fire(x4h, g3, s3)
        drain(accv, s1)
        drain(g2, s2)
        accum(g2)
        drain(g3, s3)
        fire(x5h, g2, s2)
        accum(g3)
        drain(g2, s2)
        accum(g2)
        pltpu.sync_copy(accv, outh.at[sl])

    return gather_sum


def kernel(x1, x2, x4, x5, codebook, W1):
    B = x1.shape[0]
    table = _project(codebook.T, W1)
    h = B // 2
    gs = _gather_sum(h)
    s0 = gs(x1[:h], x2[:h], x4[:h], x5[:h], table)
    s1 = gs(x1[h:], x2[h:], x4[h:], x5[h:], table)
    return jnp.concatenate([s0, s1], axis=0)
